# baseline (speedup 1.0000x reference)
import jax
import jax.numpy as jnp
from jax import lax
from jax.experimental import pallas as pl
from jax.experimental.pallas import tpu as pltpu
from jax.experimental.pallas import tpu_sc as plsc

B = 32
N = 100
GN = 1000
D = 256
ROWS = GN * B
NW = 16
RW = ROWS // NW
WIN = 72
IDXP = 2000
TOT = 1300
MROWS = 104
RBLK = 6400


def _idx_body(labels, flip, flipped, idx_out,
              flip_w, flipped_w, labels_v, idx_v, stsem):
    wid = lax.axis_index("s")
    r_base = wid * RW
    ilo = r_base >> 5
    ilo_al = pl.multiple_of(jnp.minimum(ilo & -8, GN - WIN), 8)

    d1 = pltpu.async_copy(flip.at[:, pl.ds(ilo_al, WIN)], flip_w, stsem)
    d2 = pltpu.async_copy(flipped.at[:, pl.ds(ilo_al, WIN)], flipped_w, stsem)
    d3 = pltpu.async_copy(labels, labels_v, stsem)
    d1.wait(); d2.wait(); d3.wait()

    iota = lax.iota(jnp.int32, 16)

    def idx_chunk(q, carry):
        r16 = r_base + q * 16 + iota
        b16 = r16 & 31
        i16 = jnp.minimum(r16 >> 5, GN - 1)
        fcol = i16 - ilo_al
        f16 = plsc.load_gather(flip_w, [b16, fcol])
        fl16 = plsc.load_gather(flipped_w, [b16, fcol])
        imod = i16 - 100 * ((i16 * 5243) >> 19)
        rep16 = plsc.load_gather(labels_v, [b16, imod])
        dn16 = jnp.where(f16 < 0.5, fl16, rep16)
        dn16 = jnp.minimum(jnp.maximum(dn16, 0), 90)
        idx_v[pl.ds(q * 16, 16)] = dn16
        return carry

    lax.fori_loop(0, IDXP // 16, idx_chunk, 0)
    pltpu.sync_copy(idx_v, idx_out.at[pl.ds(r_base, RW)])


def _idx_call(labels, flip, flipped):
    mesh = plsc.VectorSubcoreMesh(core_axis_name="c", subcore_axis_name="s", num_cores=1)
    return pl.kernel(
        _idx_body,
        out_type=jax.ShapeDtypeStruct((ROWS,), jnp.int32),
        mesh=mesh,
        scratch_types=[
            pltpu.VMEM((B, WIN), jnp.float32),
            pltpu.VMEM((B, WIN), jnp.int32),
            pltpu.VMEM((B, N), jnp.int32),
            pltpu.VMEM((IDXP,), jnp.int32),
            pltpu.SemaphoreType.DMA,
        ],
        compiler_params=pltpu.CompilerParams(
            use_tc_tiling_on_sc=False, needs_layout_passes=False),
    )(labels, flip, flipped)


def _rp_body(boxes2, noise2, refpts, noise_w, boxes_v, rp_v):
    wid = lax.axis_index("s")
    r_base = wid * RW
    ilo = r_base >> 5
    ilo_al = pl.multiple_of(jnp.minimum(ilo & -8, GN - WIN), 8)
    ilo_al4 = pl.multiple_of(ilo_al * 4, 32)

    pltpu.sync_copy(noise2.at[:, pl.ds(ilo_al4, WIN * 4)], noise_w)
    pltpu.sync_copy(boxes2, boxes_v)

    iota = lax.iota(jnp.int32, 16)

    def rp_chunk(q, carry):
        qq = q * 16 + iota
        rloc = qq >> 2
        k16 = qq & 3
        r16 = r_base + rloc
        b16 = r16 & 31
        i16 = r16 >> 5
        ncol = (i16 - ilo_al) * 4 + k16
        imod = i16 - 100 * ((i16 * 5243) >> 19)
        bcol = imod * 4 + k16
        n16 = plsc.load_gather(noise_w, [b16, ncol])
        bx16 = plsc.load_gather(boxes_v, [b16, bcol])
        v16 = jnp.minimum(jnp.maximum(bx16 + 0.5 * n16, 0.0), 1.0)
        rp_v[pl.ds(q * 16, 16)] = v16
        return carry

    lax.fori_loop(0, (RW * 4) // 16, rp_chunk, 0)
    pltpu.sync_copy(rp_v, refpts.at[pl.ds(wid * (RW * 4), RW * 4)])


def _rp_call(boxes2, noise2):
    mesh = plsc.VectorSubcoreMesh(core_axis_name="c", subcore_axis_name="s", num_cores=1)
    return pl.kernel(
        _rp_body,
        out_type=jax.ShapeDtypeStruct((ROWS * 4,), jnp.float32),
        mesh=mesh,
        scratch_types=[
            pltpu.VMEM((B, WIN * 4), jnp.float32),
            pltpu.VMEM((B, N * 4), jnp.float32),
            pltpu.VMEM((RW * 4,), jnp.float32),
        ],
        compiler_params=pltpu.CompilerParams(
            use_tc_tiling_on_sc=False, needs_layout_passes=False),
    )(boxes2, noise2)


def _tgt_body(idx_ref, table_ref, out_ref):
    i = pl.program_id(0)
    idx = idx_ref[pl.ds(pl.multiple_of(i * RBLK, 128), RBLK)]
    oh = (idx[:, None] == lax.broadcasted_iota(jnp.int32, (RBLK, 92), 1)
          ).astype(jnp.float32)
    out_ref[...] = jnp.dot(oh, table_ref[...],
                           preferred_element_type=jnp.float32)


def _tgt_call(idx, table):
    return pl.pallas_call(
        _tgt_body,
        grid=(ROWS // RBLK,),
        in_specs=[
            pl.BlockSpec((ROWS,), lambda i: (0,)),
            pl.BlockSpec((92, D), lambda i: (0, 0)),
        ],
        out_specs=pl.BlockSpec((RBLK, D), lambda i: (i, 0)),
        out_shape=jax.ShapeDtypeStruct((ROWS, D), jnp.float32),
    )(idx, table)


def _mask_body(nq_ref, out_ref):
    pid = pl.program_id(0)
    row = pid * MROWS + lax.broadcasted_iota(jnp.int32, (MROWS, TOT), 0)
    col = lax.broadcasted_iota(jnp.int32, (MROWS, TOT), 1)
    gr = (row * 5243) >> 19
    gc = (col * 5243) >> 19
    dn_r = row < GN
    dn_c = col < GN
    tl = jnp.logical_and(dn_r, dn_c)
    br = jnp.logical_and(jnp.logical_not(dn_r), jnp.logical_not(dn_c))
    blocked_br = nq_ref[0] < 0
    out = jnp.where(tl, jnp.where(gr != gc, 1, 0),
                    jnp.where(br, jnp.where(blocked_br, 1, 0), 1))
    out_ref[...] = out.astype(jnp.int8)


def _mask_call(nq):
    grid = (TOT + MROWS - 1) // MROWS
    return pl.pallas_call(
        _mask_body,
        grid=(grid,),
        in_specs=[pl.BlockSpec(memory_space=pltpu.SMEM)],
        out_specs=pl.BlockSpec((MROWS, TOT), lambda i: (i, 0)),
        out_shape=jax.ShapeDtypeStruct((TOT, TOT), jnp.int8),
    )(nq)


def kernel(labels, boxes, flip_mask, flipped_labels, box_noise,
           label_enc_weight, num_queries):
    labels = labels.astype(jnp.int32)
    flipped = flipped_labels.astype(jnp.int32)
    boxes2 = boxes.reshape(B, N * 4)
    noise2 = box_noise.reshape(B, GN * 4)
    idx = _idx_call(labels, flip_mask, flipped)
    rp = _rp_call(boxes2, noise2)
    tgt = _tgt_call(idx, label_enc_weight)
    nq = jnp.asarray(num_queries, jnp.int32).reshape(1)
    attn_mask = _mask_call(nq).astype(jnp.bool_)
    return tgt.reshape(GN, B, D), rp.reshape(GN, B, 4), attn_mask
